# baseline (speedup 1.0000x reference)
import functools

import jax
import jax.numpy as jnp
from jax import lax
from jax.experimental import pallas as pl
from jax.experimental.pallas import tpu as pltpu
from jax.experimental.pallas import tpu_sc as plsc

_B, _N, _D = 64, 1024, 768
_LANES = 16
_NC, _NS = 2, 16
_NW = _NC * _NS
_ROWS_W = _N // _NW
_VPR = _D // _LANES
_BB = 8
_RB = 4
_RG = _ROWS_W // _RB
_TCH = (_B // _BB) * _RG
_NBUF = 4


def _sc_body(x_hbm, pe_hbm, o_hbm, pe_v, bufs, isems, osems):
    c = lax.axis_index("c")
    s = lax.axis_index("s")
    wid = s * _NC + c
    n0 = wid * _ROWS_W
    pltpu.sync_copy(pe_hbm.at[pl.ds(n0, _ROWS_W), :], pe_v)

    def slab(t):
        bg = lax.div(t, _RG)
        rg = lax.rem(t, _RG)
        return pl.ds(bg * _BB, _BB), pl.ds(n0 + rg * _RB, _RB), rg

    def in_copy(t, j):
        bs, rs, _ = slab(t)
        return pltpu.make_async_copy(
            x_hbm.at[bs, rs, :], bufs.at[j], isems.at[j]
        )

    def out_copy(t, j):
        bs, rs, _ = slab(t)
        return pltpu.make_async_copy(
            bufs.at[j], o_hbm.at[bs, rs, :], osems.at[j]
        )

    in_copy(0, 0).start()
    in_copy(1, 1).start()

    def chunk_step(t, carry):
        j = lax.rem(t, _NBUF)
        jn = lax.rem(t + 2, _NBUF)
        _, _, rg = slab(t)
        in_copy(t, j).wait()

        @pl.when(t >= 2)
        def _():
            out_copy(t - 2, jn).wait()

        @pl.when(t + 2 < _TCH)
        def _():
            in_copy(t + 2, jn).start()

        rg8 = rg * _RB

        def row_step(r, carry2):
            pr = rg8 + r
            for c4 in range(0, _VPR, 4):
                sls = [pl.ds((c4 + k) * _LANES, _LANES) for k in range(4)]
                vals = [pe_v[pr, sl] for sl in sls]
                for bi in range(_BB):
                    for sl, a in zip(sls, vals):
                        plsc.addupdate(bufs.at[j, bi, r, sl], a)
            return carry2

        lax.fori_loop(0, _RB, row_step, 0)
        out_copy(t, j).start()
        return carry

    lax.fori_loop(0, _TCH, chunk_step, 0)
    out_copy(_TCH - 2, (_TCH - 2) % _NBUF).wait()
    out_copy(_TCH - 1, (_TCH - 1) % _NBUF).wait()


_sc_call = functools.partial(
    pl.kernel,
    out_type=jax.ShapeDtypeStruct((_B, _N, _D), jnp.float32),
    mesh=plsc.VectorSubcoreMesh(core_axis_name="c", subcore_axis_name="s"),
    scratch_types=[
        pltpu.VMEM((_ROWS_W, _D), jnp.float32),
        pltpu.VMEM((_NBUF, _BB, _RB, _D), jnp.float32),
        pltpu.SemaphoreType.DMA((_NBUF,)),
        pltpu.SemaphoreType.DMA((_NBUF,)),
    ],
    compiler_params=pltpu.CompilerParams(use_tc_tiling_on_sc=True),
)(_sc_body)


def kernel(x, pos_emb):
    return _sc_call(x, pos_emb)


# speedup vs baseline: 1.0047x; 1.0047x over previous
import functools

import jax
import jax.numpy as jnp
from jax import lax
from jax.experimental import pallas as pl
from jax.experimental.pallas import tpu as pltpu
from jax.experimental.pallas import tpu_sc as plsc

_B, _N, _D = 64, 1024, 768
_LANES = 16
_NC, _NS = 2, 16
_NW = _NC * _NS
_ROWS_W = _N // _NW
_VPR = _D // _LANES
_BB = 2
_RB = 16
_RG = _ROWS_W // _RB
_TCH = (_B // _BB) * _RG
_NBUF = 4


def _sc_body(x_hbm, pe_hbm, o_hbm, pe_v, bufs, isems, osems):
    c = lax.axis_index("c")
    s = lax.axis_index("s")
    wid = s * _NC + c
    n0 = wid * _ROWS_W
    pltpu.sync_copy(pe_hbm.at[pl.ds(n0, _ROWS_W), :], pe_v)

    def slab(t):
        bg = lax.div(t, _RG)
        rg = lax.rem(t, _RG)
        return pl.ds(bg * _BB, _BB), pl.ds(n0 + rg * _RB, _RB), rg

    def in_copy(t, j):
        bs, rs, _ = slab(t)
        return pltpu.make_async_copy(
            x_hbm.at[bs, rs, :], bufs.at[j], isems.at[j]
        )

    def out_copy(t, j):
        bs, rs, _ = slab(t)
        return pltpu.make_async_copy(
            bufs.at[j], o_hbm.at[bs, rs, :], osems.at[j]
        )

    in_copy(0, 0).start()
    in_copy(1, 1).start()

    def chunk_step(t, carry):
        j = lax.rem(t, _NBUF)
        jn = lax.rem(t + 2, _NBUF)
        _, _, rg = slab(t)
        in_copy(t, j).wait()

        @pl.when(t >= 2)
        def _():
            out_copy(t - 2, jn).wait()

        @pl.when(t + 2 < _TCH)
        def _():
            in_copy(t + 2, jn).start()

        rg8 = rg * _RB

        def row_step(r, carry2):
            pr = rg8 + r
            for c4 in range(0, _VPR, 4):
                sls = [pl.ds((c4 + k) * _LANES, _LANES) for k in range(4)]
                vals = [pe_v[pr, sl] for sl in sls]
                for bi in range(_BB):
                    for sl, a in zip(sls, vals):
                        plsc.addupdate(bufs.at[j, bi, r, sl], a)
            return carry2

        lax.fori_loop(0, _RB, row_step, 0)
        out_copy(t, j).start()
        return carry

    lax.fori_loop(0, _TCH, chunk_step, 0)
    out_copy(_TCH - 2, (_TCH - 2) % _NBUF).wait()
    out_copy(_TCH - 1, (_TCH - 1) % _NBUF).wait()


_sc_call = functools.partial(
    pl.kernel,
    out_type=jax.ShapeDtypeStruct((_B, _N, _D), jnp.float32),
    mesh=plsc.VectorSubcoreMesh(core_axis_name="c", subcore_axis_name="s"),
    scratch_types=[
        pltpu.VMEM((_ROWS_W, _D), jnp.float32),
        pltpu.VMEM((_NBUF, _BB, _RB, _D), jnp.float32),
        pltpu.SemaphoreType.DMA((_NBUF,)),
        pltpu.SemaphoreType.DMA((_NBUF,)),
    ],
    compiler_params=pltpu.CompilerParams(use_tc_tiling_on_sc=True),
)(_sc_body)


def kernel(x, pos_emb):
    return _sc_call(x, pos_emb)


# speedup vs baseline: 1.0077x; 1.0030x over previous
import functools

import jax
import jax.numpy as jnp
from jax import lax
from jax.experimental import pallas as pl
from jax.experimental.pallas import tpu as pltpu
from jax.experimental.pallas import tpu_sc as plsc

_B, _N, _D = 64, 1024, 768
_LANES = 16
_NC, _NS = 2, 16
_NW = _NC * _NS
_ROWS_W = _N // _NW
_VPR = _D // _LANES
_BB = 2
_RB = 8
_RG = _ROWS_W // _RB
_TCH = (_B // _BB) * _RG
_NBUF = 4


def _sc_body(x_hbm, pe_hbm, o_hbm, pe_v, bufs, isems, osems):
    c = lax.axis_index("c")
    s = lax.axis_index("s")
    wid = s * _NC + c
    n0 = wid * _ROWS_W
    pltpu.sync_copy(pe_hbm.at[pl.ds(n0, _ROWS_W), :], pe_v)

    def slab(t):
        bg = lax.div(t, _RG)
        rg = lax.rem(t, _RG)
        return pl.ds(bg * _BB, _BB), pl.ds(n0 + rg * _RB, _RB), rg

    def in_copy(t, j):
        bs, rs, _ = slab(t)
        return pltpu.make_async_copy(
            x_hbm.at[bs, rs, :], bufs.at[j], isems.at[j]
        )

    def out_copy(t, j):
        bs, rs, _ = slab(t)
        return pltpu.make_async_copy(
            bufs.at[j], o_hbm.at[bs, rs, :], osems.at[j]
        )

    in_copy(0, 0).start()
    in_copy(1, 1).start()

    def chunk_step(t, carry):
        j = lax.rem(t, _NBUF)
        jn = lax.rem(t + 2, _NBUF)
        _, _, rg = slab(t)
        in_copy(t, j).wait()

        @pl.when(t >= 2)
        def _():
            out_copy(t - 2, jn).wait()

        @pl.when(t + 2 < _TCH)
        def _():
            in_copy(t + 2, jn).start()

        rg8 = rg * _RB

        def row_step(r, carry2):
            pr = rg8 + r
            for c4 in range(0, _VPR, 4):
                sls = [pl.ds((c4 + k) * _LANES, _LANES) for k in range(4)]
                vals = [pe_v[pr, sl] for sl in sls]
                for bi in range(_BB):
                    for sl, a in zip(sls, vals):
                        plsc.addupdate(bufs.at[j, bi, r, sl], a)
            return carry2

        lax.fori_loop(0, _RB, row_step, 0)
        out_copy(t, j).start()
        return carry

    lax.fori_loop(0, _TCH, chunk_step, 0)
    out_copy(_TCH - 2, (_TCH - 2) % _NBUF).wait()
    out_copy(_TCH - 1, (_TCH - 1) % _NBUF).wait()


_sc_call = functools.partial(
    pl.kernel,
    out_type=jax.ShapeDtypeStruct((_B, _N, _D), jnp.float32),
    mesh=plsc.VectorSubcoreMesh(core_axis_name="c", subcore_axis_name="s"),
    scratch_types=[
        pltpu.VMEM((_ROWS_W, _D), jnp.float32),
        pltpu.VMEM((_NBUF, _BB, _RB, _D), jnp.float32),
        pltpu.SemaphoreType.DMA((_NBUF,)),
        pltpu.SemaphoreType.DMA((_NBUF,)),
    ],
    compiler_params=pltpu.CompilerParams(use_tc_tiling_on_sc=True),
)(_sc_body)


def kernel(x, pos_emb):
    return _sc_call(x, pos_emb)
